# baseline (speedup 1.0000x reference)
import functools

import jax
import jax.numpy as jnp
import numpy as np
from jax import lax
from jax.experimental import pallas as pl
from jax.experimental.pallas import tpu as pltpu
from jax.experimental.pallas import tpu_sc as plsc

B, S, D = 4, 4096, 1024
ROWS = B * S
NUM_WORKERS = 32
CHUNK = 32

SC_ROWS = 5120
SC_IN_ROWS = SC_ROWS + (8 if SC_ROWS % S else 0)
TC_ROWS = ROWS - SC_ROWS
SLAB = SC_ROWS // NUM_WORKERS
NCHUNK = SLAB // CHUNK
CGRP = D // 16

TC_BS = 512
TC_NBLK = TC_ROWS // TC_BS

_T_CHANGE = np.uint32(838860)
_T_MINUS = np.uint32(419430)


def _threefry_bits(p):
    ks0 = np.uint32(0)
    ks1 = np.uint32(42)
    ks2 = np.uint32(0 ^ 42 ^ 0x1BD11BDA)

    def rotl(x, r):
        return lax.shift_left(x, np.uint32(r)) | lax.shift_right_logical(
            x, np.uint32(32 - r))

    x0 = jnp.zeros_like(p)
    x1 = p + ks1
    rot_a = (13, 15, 26, 6)
    rot_b = (17, 29, 16, 24)
    sched = ((ks1, ks2, 1), (ks2, ks0, 2), (ks0, ks1, 3), (ks1, ks2, 4),
             (ks2, ks0, 5))
    for g, (a0, a1, c) in enumerate(sched):
        for r in (rot_a if g % 2 == 0 else rot_b):
            x0 = x0 + x1
            x1 = x0 ^ rotl(x1, r)
        x0 = x0 + a0
        x1 = x1 + np.uint32(a1 + np.uint32(c))
    return x0 ^ x1


def _sc_body(x_hbm, out_hbm, in_buf, out_buf):
    nc = 2
    wid = lax.axis_index("s") * nc + lax.axis_index("c")
    slab0 = wid * SLAB
    lane = lax.iota(jnp.uint32, 16)

    def chunk_body(chunk, _):
        g0 = slab0 + chunk * CHUNK
        is_first = lax.rem(g0, S) == 0
        is_last = lax.rem(g0 + CHUNK, S) == 0
        interior = jnp.logical_and(jnp.logical_not(is_first),
                                   jnp.logical_not(is_last))

        @pl.when(is_first)
        def _():
            pltpu.sync_copy(x_hbm.at[pl.ds(g0, CHUNK + 1)],
                            in_buf.at[pl.ds(1, CHUNK + 1)])
            pltpu.sync_copy(x_hbm.at[pl.ds(g0, 1)], in_buf.at[pl.ds(0, 1)])

        @pl.when(is_last)
        def _():
            pltpu.sync_copy(x_hbm.at[pl.ds(g0 - 1, CHUNK + 1)],
                            in_buf.at[pl.ds(0, CHUNK + 1)])
            pltpu.sync_copy(x_hbm.at[pl.ds(g0 + CHUNK - 1, 1)],
                            in_buf.at[pl.ds(CHUNK + 1, 1)])

        @pl.when(interior)
        def _():
            pltpu.sync_copy(x_hbm.at[pl.ds(g0 - 1, CHUNK + 2)],
                            in_buf.at[pl.ds(0, CHUNK + 2)])

        p_chunk0 = lax.convert_element_type(g0 * D, jnp.uint32)

        @plsc.parallel_loop(0, CHUNK * CGRP, unroll=2)
        def vec_body(i):
            s_local = lax.shift_right_logical(i, 6)
            c16 = lax.shift_left(lax.bitwise_and(i, 63), 4)
            p = (p_chunk0 + lax.convert_element_type(s_local * D + c16,
                                                     jnp.uint32)) + lane
            m = lax.shift_right_logical(_threefry_bits(p), np.uint32(9))
            v_dn = in_buf[s_local, pl.ds(c16, 16)]
            v_mid = in_buf[s_local + 1, pl.ds(c16, 16)]
            v_up = in_buf[s_local + 2, pl.ds(c16, 16)]
            moved = jnp.where(m <= _T_MINUS, v_dn, v_up)
            out_buf[s_local, pl.ds(c16, 16)] = jnp.where(
                m <= _T_CHANGE, moved, v_mid)
        pltpu.sync_copy(out_buf, out_hbm.at[pl.ds(g0, CHUNK)])
        return _

    lax.fori_loop(0, NCHUNK, chunk_body, None)


def _sc_call(x2d):
    mesh = plsc.VectorSubcoreMesh(core_axis_name="c", subcore_axis_name="s")
    return pl.kernel(
        _sc_body,
        mesh=mesh,
        out_type=jax.ShapeDtypeStruct((SC_ROWS, D), jnp.float32),
        compiler_params=pltpu.CompilerParams(use_tc_tiling_on_sc=False),
        scratch_types=[
            pltpu.VMEM((CHUNK + 2, D), jnp.float32),
            pltpu.VMEM((CHUNK, D), jnp.float32),
        ],
    )(x2d)


def _tc_body(prev_ref, mid_ref, next_ref, o_ref):
    i = pl.program_id(0)
    row0 = SC_ROWS + i * TC_BS
    mid = mid_ref[...]
    v_dn = jnp.concatenate([prev_ref[7:8, :], mid[:TC_BS - 1]], axis=0)
    v_up = jnp.concatenate([mid[1:], next_ref[0:1, :]], axis=0)
    rows = row0 + lax.broadcasted_iota(jnp.int32, (TC_BS, D), 0)
    p = (rows * D + lax.broadcasted_iota(jnp.int32, (TC_BS, D), 1))
    m = lax.shift_right_logical(
        _threefry_bits(lax.convert_element_type(p, jnp.uint32)), np.uint32(9))
    s = lax.rem(rows, S)
    take_dn = jnp.logical_and(m <= _T_MINUS, s != 0)
    take_up = jnp.logical_and(
        jnp.logical_and(m <= _T_CHANGE, m > _T_MINUS), s != S - 1)
    o_ref[...] = jnp.where(take_dn, v_dn, jnp.where(take_up, v_up, mid))


def _tc_call(x2d):
    blk0 = SC_ROWS // TC_BS
    hblk0 = SC_ROWS // 8
    nh = ROWS // 8
    return pl.pallas_call(
        _tc_body,
        grid=(TC_NBLK,),
        in_specs=[
            pl.BlockSpec((8, D),
                         lambda i: (jnp.maximum(hblk0 + i * (TC_BS // 8) - 1,
                                                0), 0)),
            pl.BlockSpec((TC_BS, D), lambda i: (blk0 + i, 0)),
            pl.BlockSpec((8, D),
                         lambda i: (jnp.minimum(
                             hblk0 + (i + 1) * (TC_BS // 8), nh - 1), 0)),
        ],
        out_specs=pl.BlockSpec((TC_BS, D), lambda i: (blk0 + i, 0)),
        out_shape=jax.ShapeDtypeStruct((ROWS, D), jnp.float32),
    )(x2d, x2d, x2d)


@jax.jit
def kernel(x):
    x2d = x.reshape(ROWS, D)
    out_sc = _sc_call(x2d[:SC_IN_ROWS])
    out_full = _tc_call(x2d)
    return lax.dynamic_update_slice(out_full, out_sc, (0, 0)).reshape(B, S, D)


# speedup vs baseline: 1.1873x; 1.1873x over previous
import functools

import jax
import jax.numpy as jnp
import numpy as np
from jax import lax
from jax.experimental import pallas as pl
from jax.experimental.pallas import tpu as pltpu
from jax.experimental.pallas import tpu_sc as plsc

B, S, D = 4, 4096, 1024
ROWS = B * S
NUM_WORKERS = 32
CHUNK = 32

SC_ROWS = 4096
SC_IN_ROWS = SC_ROWS + (8 if SC_ROWS % S else 0)
TC_ROWS = ROWS - SC_ROWS
SLAB = SC_ROWS // NUM_WORKERS
NCHUNK = SLAB // CHUNK
CGRP = D // 16

TC_BS = 512
TC_NBLK = TC_ROWS // TC_BS

_T_CHANGE = np.uint32(838860)
_T_MINUS = np.uint32(419430)


def _threefry_bits(p):
    ks0 = np.uint32(0)
    ks1 = np.uint32(42)
    ks2 = np.uint32(0 ^ 42 ^ 0x1BD11BDA)

    def rotl(x, r):
        return lax.shift_left(x, np.uint32(r)) | lax.shift_right_logical(
            x, np.uint32(32 - r))

    x0 = jnp.zeros_like(p)
    x1 = p + ks1
    rot_a = (13, 15, 26, 6)
    rot_b = (17, 29, 16, 24)
    sched = ((ks1, ks2, 1), (ks2, ks0, 2), (ks0, ks1, 3), (ks1, ks2, 4),
             (ks2, ks0, 5))
    for g, (a0, a1, c) in enumerate(sched):
        for r in (rot_a if g % 2 == 0 else rot_b):
            x0 = x0 + x1
            x1 = x0 ^ rotl(x1, r)
        x0 = x0 + a0
        x1 = x1 + np.uint32(a1 + np.uint32(c))
    return x0 ^ x1


def _sc_body(x_hbm, out_hbm, in_buf, out_buf):
    nc = 2
    wid = lax.axis_index("s") * nc + lax.axis_index("c")
    slab0 = wid * SLAB
    lane = lax.iota(jnp.uint32, 16)

    def chunk_body(chunk, _):
        g0 = slab0 + chunk * CHUNK
        is_first = lax.rem(g0, S) == 0
        is_last = lax.rem(g0 + CHUNK, S) == 0
        interior = jnp.logical_and(jnp.logical_not(is_first),
                                   jnp.logical_not(is_last))

        @pl.when(is_first)
        def _():
            pltpu.sync_copy(x_hbm.at[pl.ds(g0, CHUNK + 1)],
                            in_buf.at[pl.ds(1, CHUNK + 1)])
            pltpu.sync_copy(x_hbm.at[pl.ds(g0, 1)], in_buf.at[pl.ds(0, 1)])

        @pl.when(is_last)
        def _():
            pltpu.sync_copy(x_hbm.at[pl.ds(g0 - 1, CHUNK + 1)],
                            in_buf.at[pl.ds(0, CHUNK + 1)])
            pltpu.sync_copy(x_hbm.at[pl.ds(g0 + CHUNK - 1, 1)],
                            in_buf.at[pl.ds(CHUNK + 1, 1)])

        @pl.when(interior)
        def _():
            pltpu.sync_copy(x_hbm.at[pl.ds(g0 - 1, CHUNK + 2)],
                            in_buf.at[pl.ds(0, CHUNK + 2)])

        p_chunk0 = lax.convert_element_type(g0 * D, jnp.uint32)

        @plsc.parallel_loop(0, CHUNK * CGRP, unroll=2)
        def vec_body(i):
            s_local = lax.shift_right_logical(i, 6)
            c16 = lax.shift_left(lax.bitwise_and(i, 63), 4)
            p = (p_chunk0 + lax.convert_element_type(s_local * D + c16,
                                                     jnp.uint32)) + lane
            m = lax.shift_right_logical(_threefry_bits(p), np.uint32(9))
            v_dn = in_buf[s_local, pl.ds(c16, 16)]
            v_mid = in_buf[s_local + 1, pl.ds(c16, 16)]
            v_up = in_buf[s_local + 2, pl.ds(c16, 16)]
            moved = jnp.where(m <= _T_MINUS, v_dn, v_up)
            out_buf[s_local, pl.ds(c16, 16)] = jnp.where(
                m <= _T_CHANGE, moved, v_mid)
        pltpu.sync_copy(out_buf, out_hbm.at[pl.ds(g0, CHUNK)])
        return _

    lax.fori_loop(0, NCHUNK, chunk_body, None)


def _sc_call(x2d):
    mesh = plsc.VectorSubcoreMesh(core_axis_name="c", subcore_axis_name="s")
    return pl.kernel(
        _sc_body,
        mesh=mesh,
        out_type=jax.ShapeDtypeStruct((SC_ROWS, D), jnp.float32),
        compiler_params=pltpu.CompilerParams(use_tc_tiling_on_sc=False),
        scratch_types=[
            pltpu.VMEM((CHUNK + 2, D), jnp.float32),
            pltpu.VMEM((CHUNK, D), jnp.float32),
        ],
    )(x2d)


def _tc_body(prev_ref, mid_ref, next_ref, o_ref):
    i = pl.program_id(0)
    row0 = SC_ROWS + i * TC_BS
    mid = mid_ref[...]
    v_dn = jnp.concatenate([prev_ref[7:8, :], mid[:TC_BS - 1]], axis=0)
    v_up = jnp.concatenate([mid[1:], next_ref[0:1, :]], axis=0)
    rows = row0 + lax.broadcasted_iota(jnp.int32, (TC_BS, D), 0)
    p = (rows * D + lax.broadcasted_iota(jnp.int32, (TC_BS, D), 1))
    m = lax.shift_right_logical(
        _threefry_bits(lax.convert_element_type(p, jnp.uint32)), np.uint32(9))
    s = lax.rem(rows, S)
    take_dn = jnp.logical_and(m <= _T_MINUS, s != 0)
    take_up = jnp.logical_and(
        jnp.logical_and(m <= _T_CHANGE, m > _T_MINUS), s != S - 1)
    o_ref[...] = jnp.where(take_dn, v_dn, jnp.where(take_up, v_up, mid))


def _tc_call(x2d):
    blk0 = SC_ROWS // TC_BS
    hblk0 = SC_ROWS // 8
    nh = ROWS // 8
    return pl.pallas_call(
        _tc_body,
        grid=(TC_NBLK,),
        in_specs=[
            pl.BlockSpec((8, D),
                         lambda i: (jnp.maximum(hblk0 + i * (TC_BS // 8) - 1,
                                                0), 0)),
            pl.BlockSpec((TC_BS, D), lambda i: (blk0 + i, 0)),
            pl.BlockSpec((8, D),
                         lambda i: (jnp.minimum(
                             hblk0 + (i + 1) * (TC_BS // 8), nh - 1), 0)),
        ],
        out_specs=pl.BlockSpec((TC_BS, D), lambda i: (blk0 + i, 0)),
        out_shape=jax.ShapeDtypeStruct((ROWS, D), jnp.float32),
    )(x2d, x2d, x2d)


@jax.jit
def kernel(x):
    x2d = x.reshape(ROWS, D)
    out_sc = _sc_call(x2d[:SC_IN_ROWS])
    out_full = _tc_call(x2d)
    return lax.dynamic_update_slice(out_full, out_sc, (0, 0)).reshape(B, S, D)


# speedup vs baseline: 1.1883x; 1.0008x over previous
import jax
import jax.numpy as jnp
import numpy as np
from jax import lax
from jax.experimental import pallas as pl
from jax.experimental.pallas import tpu as pltpu
from jax.experimental.pallas import tpu_sc as plsc

B, S, D = 4, 4096, 1024
ROWS = B * S
NUM_WORKERS = 32
CHUNK = 32

SC_ROWS = 4096
SC_IN_ROWS = SC_ROWS + (8 if SC_ROWS % S else 0)
TC_ROWS = ROWS - SC_ROWS
SLAB = SC_ROWS // NUM_WORKERS
NCHUNK = SLAB // CHUNK
CGRP = D // 16

TC_BS = 512
TC_NBLK = TC_ROWS // TC_BS

_T_CHANGE = np.uint32(838860)
_T_MINUS = np.uint32(419430)


def _threefry_bits(p):
    ks0 = np.uint32(0)
    ks1 = np.uint32(42)
    ks2 = np.uint32(0 ^ 42 ^ 0x1BD11BDA)

    def rotl(x, r):
        return lax.shift_left(x, np.uint32(r)) | lax.shift_right_logical(
            x, np.uint32(32 - r))

    x0 = jnp.zeros_like(p)
    x1 = p + ks1
    rot_a = (13, 15, 26, 6)
    rot_b = (17, 29, 16, 24)
    sched = ((ks1, ks2, 1), (ks2, ks0, 2), (ks0, ks1, 3), (ks1, ks2, 4),
             (ks2, ks0, 5))
    for g, (a0, a1, c) in enumerate(sched):
        for r in (rot_a if g % 2 == 0 else rot_b):
            x0 = x0 + x1
            x1 = x0 ^ rotl(x1, r)
        x0 = x0 + a0
        x1 = x1 + np.uint32(a1 + np.uint32(c))
    return x0 ^ x1


def _sc_body(x_hbm, out_hbm, in_buf, out_buf):
    nc = 2
    wid = lax.axis_index("s") * nc + lax.axis_index("c")
    slab0 = wid * SLAB
    lane = lax.iota(jnp.uint32, 16)

    def chunk_body(chunk, _):
        g0 = slab0 + chunk * CHUNK
        is_first = lax.rem(g0, S) == 0
        is_last = lax.rem(g0 + CHUNK, S) == 0
        interior = jnp.logical_and(jnp.logical_not(is_first),
                                   jnp.logical_not(is_last))

        @pl.when(is_first)
        def _():
            pltpu.sync_copy(x_hbm.at[pl.ds(g0, CHUNK + 1)],
                            in_buf.at[pl.ds(1, CHUNK + 1)])
            pltpu.sync_copy(x_hbm.at[pl.ds(g0, 1)], in_buf.at[pl.ds(0, 1)])

        @pl.when(is_last)
        def _():
            pltpu.sync_copy(x_hbm.at[pl.ds(g0 - 1, CHUNK + 1)],
                            in_buf.at[pl.ds(0, CHUNK + 1)])
            pltpu.sync_copy(x_hbm.at[pl.ds(g0 + CHUNK - 1, 1)],
                            in_buf.at[pl.ds(CHUNK + 1, 1)])

        @pl.when(interior)
        def _():
            pltpu.sync_copy(x_hbm.at[pl.ds(g0 - 1, CHUNK + 2)],
                            in_buf.at[pl.ds(0, CHUNK + 2)])

        p_chunk0 = lax.convert_element_type(g0 * D, jnp.uint32)

        @plsc.parallel_loop(0, CHUNK * CGRP, unroll=2)
        def vec_body(i):
            s_local = lax.shift_right_logical(i, 6)
            c16 = lax.shift_left(lax.bitwise_and(i, 63), 4)
            p = (p_chunk0 + lax.convert_element_type(s_local * D + c16,
                                                     jnp.uint32)) + lane
            m = lax.shift_right_logical(_threefry_bits(p), np.uint32(9))
            v_dn = in_buf[s_local, pl.ds(c16, 16)]
            v_mid = in_buf[s_local + 1, pl.ds(c16, 16)]
            v_up = in_buf[s_local + 2, pl.ds(c16, 16)]
            moved = jnp.where(m <= _T_MINUS, v_dn, v_up)
            out_buf[s_local, pl.ds(c16, 16)] = jnp.where(
                m <= _T_CHANGE, moved, v_mid)
        pltpu.sync_copy(out_buf, out_hbm.at[pl.ds(g0, CHUNK)])
        return _

    lax.fori_loop(0, NCHUNK, chunk_body, None)


def _sc_call(x2d):
    mesh = plsc.VectorSubcoreMesh(core_axis_name="c", subcore_axis_name="s")
    return pl.kernel(
        _sc_body,
        mesh=mesh,
        out_type=jax.ShapeDtypeStruct((SC_ROWS, D), jnp.float32),
        compiler_params=pltpu.CompilerParams(use_tc_tiling_on_sc=False),
        scratch_types=[
            pltpu.VMEM((CHUNK + 2, D), jnp.float32),
            pltpu.VMEM((CHUNK, D), jnp.float32),
        ],
    )(x2d)


def _tc_body(prev_ref, mid_ref, next_ref, o_ref):
    i = pl.program_id(0)
    row0 = SC_ROWS + i * TC_BS
    mid = mid_ref[...]
    v_dn = jnp.concatenate([prev_ref[7:8, :], mid[:TC_BS - 1]], axis=0)
    v_up = jnp.concatenate([mid[1:], next_ref[0:1, :]], axis=0)
    rows = row0 + lax.broadcasted_iota(jnp.int32, (TC_BS, D), 0)
    p = (rows * D + lax.broadcasted_iota(jnp.int32, (TC_BS, D), 1))
    m = lax.shift_right_logical(
        _threefry_bits(lax.convert_element_type(p, jnp.uint32)), np.uint32(9))
    s = lax.rem(rows, S)
    take_dn = jnp.logical_and(m <= _T_MINUS, s != 0)
    take_up = jnp.logical_and(
        jnp.logical_and(m <= _T_CHANGE, m > _T_MINUS), s != S - 1)
    o_ref[...] = jnp.where(take_dn, v_dn, jnp.where(take_up, v_up, mid))


def _tc_call(x2d):
    blk0 = SC_ROWS // TC_BS
    hblk0 = SC_ROWS // 8
    nh = ROWS // 8
    return pl.pallas_call(
        _tc_body,
        grid=(TC_NBLK,),
        in_specs=[
            pl.BlockSpec((8, D),
                         lambda i: (jnp.maximum(hblk0 + i * (TC_BS // 8) - 1,
                                                0), 0)),
            pl.BlockSpec((TC_BS, D), lambda i: (blk0 + i, 0)),
            pl.BlockSpec((8, D),
                         lambda i: (jnp.minimum(
                             hblk0 + (i + 1) * (TC_BS // 8), nh - 1), 0)),
        ],
        out_specs=pl.BlockSpec((TC_BS, D), lambda i: (blk0 + i, 0)),
        out_shape=jax.ShapeDtypeStruct((ROWS, D), jnp.float32),
    )(x2d, x2d, x2d)


@jax.jit
def kernel(x):
    x2d = x.reshape(ROWS, D)
    out_sc = _sc_call(x2d[:SC_IN_ROWS])
    out_full = _tc_call(x2d)
    return lax.dynamic_update_slice(out_full, out_sc, (0, 0)).reshape(B, S, D)
